# baseline (speedup 1.0000x reference)
import jax
import jax.numpy as jnp
from jax import lax
from jax.experimental import pallas as pl
from jax.experimental.pallas import tpu as pltpu
from jax.experimental.pallas import tpu_sc as plsc

_HIDDEN = 1024
_INTER = 2048
_E = 8
_TOPK = 2
_B = 256
_NW = 32
_L = 16


def _router_body(hs_ref, rw_ref, da_ref, db_ref, wa_ref, wb_ref, cend_ref):
    T = hs_ref.shape[0]
    P = _TOPK * T
    g = jax.lax.dot_general(
        hs_ref[...], rw_ref[...], (((1,), (1,)), ((), ())),
        preferred_element_type=jnp.float32)
    ii = jax.lax.broadcasted_iota(jnp.int32, g.shape, 1)
    m1 = jnp.max(g, axis=1, keepdims=True)
    e1 = jnp.min(jnp.where(g >= m1, ii, _E), axis=1, keepdims=True)
    s = jnp.sum(jnp.exp(g - m1), axis=1, keepdims=True)
    g2 = jnp.where(ii == e1, -jnp.inf, g)
    m2 = jnp.max(g2, axis=1, keepdims=True)
    e2 = jnp.min(jnp.where(g2 >= m2, ii, _E), axis=1, keepdims=True)
    wa_ref[...] = jnp.broadcast_to(1.0 / s, (T, _L))
    wb_ref[...] = jnp.broadcast_to(jnp.exp(m2 - m1) / s, (T, _L))

    fe = jnp.concatenate([e1, e2], axis=0)
    oh = (fe == jax.lax.broadcasted_iota(jnp.int32, (P, _E), 1)).astype(
        jnp.int32)
    incl = oh
    k = 1
    while k < P:
        incl = incl + jnp.concatenate(
            [jnp.zeros((k, _E), jnp.int32), incl[:P - k]], axis=0)
        k *= 2
    counts = incl[P - 1:P, :]
    rank = jnp.sum(jnp.where(oh == 1, incl, 0), axis=1, keepdims=True) - 1
    nblk = (counts + _B - 1) // _B
    cend = nblk
    k = 1
    while k < _E:
        cend = cend + jnp.concatenate(
            [jnp.zeros((1, k), jnp.int32), cend[:, :_E - k]], axis=1)
        k *= 2
    blk_start = cend - nblk
    bs = jnp.sum(jnp.where(oh == 1, jnp.broadcast_to(blk_start, (P, _E)), 0),
                 axis=1, keepdims=True)
    dest = bs * _B + rank
    da_ref[...] = dest[:T]
    db_ref[...] = dest[T:]
    cend_ref[...] = jnp.broadcast_to(cend, (_E, _E))


def _ffn_body(s_ref, nact_ref, x_ref, w1_ref, w2_ref, y_ref):
    del s_ref
    b = pl.program_id(0)

    @pl.when(b < nact_ref[0])
    def _():
        x = x_ref[...].astype(jnp.bfloat16)
        h = jax.lax.dot_general(
            x, w1_ref[0].astype(jnp.bfloat16), (((1,), (1,)), ((), ())),
            preferred_element_type=jnp.float32)
        gate = h[:, :_INTER]
        up = h[:, _INTER:]
        a = (up * (gate * jax.nn.sigmoid(gate))).astype(jnp.bfloat16)
        y_ref[...] = jax.lax.dot_general(
            a, w2_ref[0].astype(jnp.bfloat16), (((1,), (1,)), ((), ())),
            preferred_element_type=jnp.float32)


def _gather_body(hs_hbm, da_hbm, db_hbm, x_hbm, rows_v, idx_v, sem):
    wid = lax.axis_index("s") * 2 + lax.axis_index("c")
    base = wid * 64
    pltpu.sync_copy(hs_hbm.at[pl.ds(base, 64)], rows_v)
    pltpu.sync_copy(da_hbm.at[pl.ds(base, 64)], idx_v)
    pltpu.async_copy(rows_v, x_hbm.at[idx_v], sem).wait()
    pltpu.sync_copy(db_hbm.at[pl.ds(base, 64)], idx_v)
    pltpu.async_copy(rows_v, x_hbm.at[idx_v], sem).wait()


def _dgather_body(y_hbm, da_hbm, db_hbm, yab_hbm, buf, idx_v, sem):
    wid = lax.axis_index("s") * 2 + lax.axis_index("c")
    base = wid * 64
    T = da_hbm.shape[0]
    pltpu.sync_copy(da_hbm.at[pl.ds(base, 64)], idx_v)
    pltpu.async_copy(y_hbm.at[idx_v], buf, sem).wait()
    pltpu.sync_copy(buf, yab_hbm.at[pl.ds(base, 64)])
    pltpu.sync_copy(db_hbm.at[pl.ds(base, 64)], idx_v)
    pltpu.async_copy(y_hbm.at[idx_v], buf, sem).wait()
    pltpu.sync_copy(buf, yab_hbm.at[pl.ds(T + base, 64)])


def _mix_body(ya_ref, yb_ref, wa_ref, wb_ref, o_ref):
    wa = wa_ref[:, :1]
    wb = wb_ref[:, :1]
    o_ref[...] = wa * ya_ref[...] + wb * yb_ref[...]


def kernel(hidden_states, w1, w2, router_w):
    orig_shape = hidden_states.shape
    hs = hidden_states.reshape(-1, _HIDDEN)
    T = hs.shape[0]
    P = _TOPK * T
    NB = P // _B + _E
    PAD = NB * _B

    da, db, wab, wbb, cendb = pl.pallas_call(
        _router_body,
        out_shape=[
            jax.ShapeDtypeStruct((T, 1), jnp.int32),
            jax.ShapeDtypeStruct((T, 1), jnp.int32),
            jax.ShapeDtypeStruct((T, _L), jnp.float32),
            jax.ShapeDtypeStruct((T, _L), jnp.float32),
            jax.ShapeDtypeStruct((_E, _E), jnp.int32),
        ],
    )(hs, router_w)
    da = da.reshape(T)
    db = db.reshape(T)
    cend = cendb[0]
    block_expert = jnp.minimum(
        jnp.sum(jnp.arange(NB, dtype=jnp.int32)[:, None] >= cend[None, :],
                axis=1), _E - 1).astype(jnp.int32)
    nact = cend[_E - 1:]

    X = pl.kernel(
        _gather_body,
        mesh=plsc.VectorSubcoreMesh(core_axis_name="c", subcore_axis_name="s"),
        out_type=jax.ShapeDtypeStruct((PAD, _HIDDEN), jnp.float32),
        scratch_types=[
            pltpu.VMEM((64, _HIDDEN), jnp.float32),
            pltpu.VMEM((64,), jnp.int32),
            pltpu.SemaphoreType.DMA,
        ],
    )(hs, da, db)

    grid_spec = pltpu.PrefetchScalarGridSpec(
        num_scalar_prefetch=2,
        grid=(NB,),
        in_specs=[
            pl.BlockSpec((_B, _HIDDEN), lambda b, s, n: (b, 0)),
            pl.BlockSpec((1, 2 * _INTER, _HIDDEN), lambda b, s, n: (s[b], 0, 0)),
            pl.BlockSpec((1, _HIDDEN, _INTER), lambda b, s, n: (s[b], 0, 0)),
        ],
        out_specs=pl.BlockSpec((_B, _HIDDEN), lambda b, s, n: (b, 0)),
    )
    Y = pl.pallas_call(
        _ffn_body,
        grid_spec=grid_spec,
        out_shape=jax.ShapeDtypeStruct((PAD, _HIDDEN), jnp.float32),
    )(block_expert, nact, X, w1, w2)

    Yab = pl.kernel(
        _dgather_body,
        mesh=plsc.VectorSubcoreMesh(core_axis_name="c", subcore_axis_name="s"),
        out_type=jax.ShapeDtypeStruct((2 * T, _HIDDEN), jnp.float32),
        scratch_types=[
            pltpu.VMEM((64, _HIDDEN), jnp.float32),
            pltpu.VMEM((64,), jnp.int32),
            pltpu.SemaphoreType.DMA,
        ],
    )(Y, da, db)
    BC = 512
    NC = T // BC
    out = pl.pallas_call(
        _mix_body,
        grid=(NC,),
        in_specs=[
            pl.BlockSpec((BC, _HIDDEN), lambda i: (i, 0)),
            pl.BlockSpec((BC, _HIDDEN), lambda i: (i + NC, 0)),
            pl.BlockSpec((BC, _L), lambda i: (i, 0)),
            pl.BlockSpec((BC, _L), lambda i: (i, 0)),
        ],
        out_specs=pl.BlockSpec((BC, _HIDDEN), lambda i: (i, 0)),
        out_shape=jax.ShapeDtypeStruct((T, _HIDDEN), jnp.float32),
    )(Yab, Yab, wab, wbb)
    return out.reshape(orig_shape)


# speedup vs baseline: 1.0045x; 1.0045x over previous
import jax
import jax.numpy as jnp
from jax import lax
from jax.experimental import pallas as pl
from jax.experimental.pallas import tpu as pltpu
from jax.experimental.pallas import tpu_sc as plsc

_HIDDEN = 1024
_INTER = 2048
_E = 8
_TOPK = 2
_B = 256
_NW = 32
_L = 16


def _router_body(hs_ref, rw_ref, da_ref, db_ref, wa_ref, wb_ref, cend_ref):
    T = hs_ref.shape[0]
    P = _TOPK * T
    g = jax.lax.dot_general(
        hs_ref[...], rw_ref[...], (((1,), (1,)), ((), ())),
        preferred_element_type=jnp.float32)
    ii = jax.lax.broadcasted_iota(jnp.int32, g.shape, 1)
    m1 = jnp.max(g, axis=1, keepdims=True)
    e1 = jnp.min(jnp.where(g >= m1, ii, _E), axis=1, keepdims=True)
    s = jnp.sum(jnp.exp(g - m1), axis=1, keepdims=True)
    g2 = jnp.where(ii == e1, -jnp.inf, g)
    m2 = jnp.max(g2, axis=1, keepdims=True)
    e2 = jnp.min(jnp.where(g2 >= m2, ii, _E), axis=1, keepdims=True)
    wa_ref[...] = jnp.broadcast_to(1.0 / s, (T, _L))
    wb_ref[...] = jnp.broadcast_to(jnp.exp(m2 - m1) / s, (T, _L))

    fe = jnp.concatenate([e1, e2], axis=0)
    oh = (fe == jax.lax.broadcasted_iota(jnp.int32, (P, _E), 1)).astype(
        jnp.int32)
    incl = oh
    k = 1
    while k < P:
        incl = incl + jnp.concatenate(
            [jnp.zeros((k, _E), jnp.int32), incl[:P - k]], axis=0)
        k *= 2
    counts = incl[P - 1:P, :]
    rank = jnp.sum(jnp.where(oh == 1, incl, 0), axis=1, keepdims=True) - 1
    nblk = (counts + _B - 1) // _B
    cend = nblk
    k = 1
    while k < _E:
        cend = cend + jnp.concatenate(
            [jnp.zeros((1, k), jnp.int32), cend[:, :_E - k]], axis=1)
        k *= 2
    blk_start = cend - nblk
    bs = jnp.sum(jnp.where(oh == 1, jnp.broadcast_to(blk_start, (P, _E)), 0),
                 axis=1, keepdims=True)
    dest = bs * _B + rank
    da_ref[...] = dest[:T]
    db_ref[...] = dest[T:]
    cend_ref[...] = jnp.broadcast_to(cend, (_E, _E))


def _ffn_body(s_ref, nact_ref, x_ref, w1_ref, w2_ref, y_ref):
    del s_ref
    b = pl.program_id(0)

    @pl.when(b < nact_ref[0])
    def _():
        x = x_ref[...]
        h = jax.lax.dot_general(
            x, w1_ref[0], (((1,), (1,)), ((), ())),
            preferred_element_type=jnp.float32)
        gate = h[:, :_INTER]
        up = h[:, _INTER:]
        a = up * (gate * jax.nn.sigmoid(gate))
        y_ref[...] = jax.lax.dot_general(
            a, w2_ref[0], (((1,), (1,)), ((), ())),
            preferred_element_type=jnp.float32)


def _gather_body(hs_hbm, da_hbm, db_hbm, x_hbm, rows_v, idx_v, sem):
    wid = lax.axis_index("s") * 2 + lax.axis_index("c")
    base = wid * 64
    pltpu.sync_copy(hs_hbm.at[pl.ds(base, 64)], rows_v)
    pltpu.sync_copy(da_hbm.at[pl.ds(base, 64)], idx_v)
    pltpu.async_copy(rows_v, x_hbm.at[idx_v], sem).wait()
    pltpu.sync_copy(db_hbm.at[pl.ds(base, 64)], idx_v)
    pltpu.async_copy(rows_v, x_hbm.at[idx_v], sem).wait()


def _dgather_body(y_hbm, da_hbm, db_hbm, yab_hbm, buf, idx_v, sem):
    wid = lax.axis_index("s") * 2 + lax.axis_index("c")
    base = wid * 64
    T = da_hbm.shape[0]
    pltpu.sync_copy(da_hbm.at[pl.ds(base, 64)], idx_v)
    pltpu.async_copy(y_hbm.at[idx_v], buf, sem).wait()
    pltpu.sync_copy(buf, yab_hbm.at[pl.ds(base, 64)])
    pltpu.sync_copy(db_hbm.at[pl.ds(base, 64)], idx_v)
    pltpu.async_copy(y_hbm.at[idx_v], buf, sem).wait()
    pltpu.sync_copy(buf, yab_hbm.at[pl.ds(T + base, 64)])


def _mix_body(ya_ref, yb_ref, wa_ref, wb_ref, o_ref):
    wa = wa_ref[:, :1]
    wb = wb_ref[:, :1]
    o_ref[...] = wa * ya_ref[...] + wb * yb_ref[...]


def kernel(hidden_states, w1, w2, router_w):
    orig_shape = hidden_states.shape
    hs = hidden_states.reshape(-1, _HIDDEN)
    T = hs.shape[0]
    P = _TOPK * T
    NB = P // _B + _E
    PAD = NB * _B

    da, db, wab, wbb, cendb = pl.pallas_call(
        _router_body,
        out_shape=[
            jax.ShapeDtypeStruct((T, 1), jnp.int32),
            jax.ShapeDtypeStruct((T, 1), jnp.int32),
            jax.ShapeDtypeStruct((T, _L), jnp.float32),
            jax.ShapeDtypeStruct((T, _L), jnp.float32),
            jax.ShapeDtypeStruct((_E, _E), jnp.int32),
        ],
    )(hs, router_w)
    da = da.reshape(T)
    db = db.reshape(T)
    cend = cendb[0]
    block_expert = jnp.minimum(
        jnp.sum(jnp.arange(NB, dtype=jnp.int32)[:, None] >= cend[None, :],
                axis=1), _E - 1).astype(jnp.int32)
    nact = cend[_E - 1:]

    X = pl.kernel(
        _gather_body,
        mesh=plsc.VectorSubcoreMesh(core_axis_name="c", subcore_axis_name="s"),
        out_type=jax.ShapeDtypeStruct((PAD, _HIDDEN), jnp.float32),
        scratch_types=[
            pltpu.VMEM((64, _HIDDEN), jnp.float32),
            pltpu.VMEM((64,), jnp.int32),
            pltpu.SemaphoreType.DMA,
        ],
    )(hs, da, db)

    grid_spec = pltpu.PrefetchScalarGridSpec(
        num_scalar_prefetch=2,
        grid=(NB,),
        in_specs=[
            pl.BlockSpec((_B, _HIDDEN), lambda b, s, n: (b, 0)),
            pl.BlockSpec((1, 2 * _INTER, _HIDDEN), lambda b, s, n: (s[b], 0, 0)),
            pl.BlockSpec((1, _HIDDEN, _INTER), lambda b, s, n: (s[b], 0, 0)),
        ],
        out_specs=pl.BlockSpec((_B, _HIDDEN), lambda b, s, n: (b, 0)),
    )
    Y = pl.pallas_call(
        _ffn_body,
        grid_spec=grid_spec,
        out_shape=jax.ShapeDtypeStruct((PAD, _HIDDEN), jnp.float32),
    )(block_expert, nact, X, w1, w2)

    Yab = pl.kernel(
        _dgather_body,
        mesh=plsc.VectorSubcoreMesh(core_axis_name="c", subcore_axis_name="s"),
        out_type=jax.ShapeDtypeStruct((2 * T, _HIDDEN), jnp.float32),
        scratch_types=[
            pltpu.VMEM((64, _HIDDEN), jnp.float32),
            pltpu.VMEM((64,), jnp.int32),
            pltpu.SemaphoreType.DMA,
        ],
    )(Y, da, db)
    BC = 512
    NC = T // BC
    out = pl.pallas_call(
        _mix_body,
        grid=(NC,),
        in_specs=[
            pl.BlockSpec((BC, _HIDDEN), lambda i: (i, 0)),
            pl.BlockSpec((BC, _HIDDEN), lambda i: (i + NC, 0)),
            pl.BlockSpec((BC, _L), lambda i: (i, 0)),
            pl.BlockSpec((BC, _L), lambda i: (i, 0)),
        ],
        out_specs=pl.BlockSpec((BC, _HIDDEN), lambda i: (i, 0)),
        out_shape=jax.ShapeDtypeStruct((T, _HIDDEN), jnp.float32),
    )(Yab, Yab, wab, wbb)
    return out.reshape(orig_shape)
